# baseline (speedup 1.0000x reference)
import functools

import jax
import jax.numpy as jnp
from jax import lax
from jax.experimental import pallas as pl
from jax.experimental.pallas import tpu as pltpu
from jax.experimental.pallas import tpu_sc as plsc

EPS = 1e-9


def _dense_body(ind1_ref, ind2_ref, ind3_ref, ind4_ref, ind5_ref,
                ut_ref, item_ref, skill_ref, q_ref,
                wstu, a1, wisk, a3, wski, a4, wisu, a2,
                ams, amk,
                fu_ref, fsk_ref, packed_ref,
                v1aug_s, skf_s, acc_s):
    pid = pl.program_id(0)
    nsteps = pl.num_programs(0)
    ind_refs = (ind1_ref, ind2_ref, ind3_ref, ind4_ref, ind5_ref)
    hb = ind1_ref.shape[0]
    dnT = (((1,), (1,)), ((), ()))

    @pl.when(pid == 0)
    def _():
        it = item_ref[...]
        sk = skill_ref[...]
        qm = q_ref[...]
        ti = lax.dot_general(it, wstu[...], dnT,
                             preferred_element_type=jnp.float32)
        w1 = jnp.exp(jnp.sum(ti * a1[:, 128:], axis=1, keepdims=True))
        v1aug_s[:, :128] = (w1 * ti).astype(jnp.bfloat16)
        v1aug_s[:, 128:] = jnp.broadcast_to(w1, ti.shape).astype(jnp.bfloat16)
        tsk = lax.dot_general(sk, wisk[...], dnT,
                              preferred_element_type=jnp.float32)
        w3 = jnp.exp(jnp.sum(tsk * a3[:, 128:], axis=1, keepdims=True))
        num3 = jnp.dot(qm, w3 * tsk, preferred_element_type=jnp.float32)
        den3 = jnp.dot(qm, jnp.broadcast_to(w3, tsk.shape),
                       preferred_element_type=jnp.float32)[:, 0:1]
        skf_s[...] = num3 / (den3 + EPS)
        tis = lax.dot_general(it, wski[...], dnT,
                              preferred_element_type=jnp.float32)
        w4 = jnp.exp(jnp.sum(tis * a4[:, 128:], axis=1, keepdims=True))
        dn0 = (((0,), (0,)), ((), ()))
        num4 = lax.dot_general(qm, w4 * tis, dn0,
                               preferred_element_type=jnp.float32)
        den4 = lax.dot_general(qm, jnp.broadcast_to(w4, tis.shape), dn0,
                               preferred_element_type=jnp.float32)[:, 0:1]
        fsk_ref[...] = sk + num4 / (den4 + EPS)

    ut = ut_ref[...]
    v1 = v1aug_s[...]
    tsu = lax.dot_general(ut, wisu[...], dnT,
                          preferred_element_type=jnp.float32)
    w2 = jnp.exp(jnp.sum(tsu * a2[:, 128:], axis=1, keepdims=True))
    u2aug = jnp.concatenate(
        [w2 * tsu, jnp.broadcast_to(w2, tsu.shape)], axis=1
    ).astype(jnp.bfloat16)
    dn = (((0,), (0,)), ((), ()))
    contrib = jnp.zeros_like(acc_s)
    for j, ind_ref in enumerate(ind_refs):
        ind = ind_ref[...].astype(jnp.bfloat16)
        nd = jnp.dot(ind, v1, preferred_element_type=jnp.float32)
        fu_ref[j * hb:(j + 1) * hb, :] = (
            ut[j * hb:(j + 1) * hb, :]
            + nd[:, :128] / (nd[:, 128:129] + EPS))
        contrib = contrib + lax.dot_general(
            ind, u2aug[j * hb:(j + 1) * hb, :], dn,
            preferred_element_type=jnp.float32)

    @pl.when(pid == 0)
    def _():
        acc_s[...] = contrib

    @pl.when(pid != 0)
    def _():
        acc_s[...] = acc_s[...] + contrib

    @pl.when(pid == nsteps - 1)
    def _():
        acc = acc_s[...]
        it = item_ref[...]
        skf = skf_s[...]
        stu = acc[:, :128] / (acc[:, 128:129] + EPS)
        ms = (jnp.sum(it * ams[:, :128], axis=1, keepdims=True)
              + jnp.sum(stu * ams[:, 128:], axis=1, keepdims=True))
        mk = (jnp.sum(it * amk[:, :128], axis=1, keepdims=True)
              + jnp.sum(skf * amk[:, 128:], axis=1, keepdims=True))
        m = jnp.maximum(ms, mk)
        es = jnp.exp(ms - m)
        ek = jnp.exp(mk - m)
        tot = es + ek
        packed_ref[:, :128] = it + (es / tot) * stu + (ek / tot) * skf
        packed_ref[:, 128:] = q_ref[...]


def _dense(indicator, user_t, item_t, skill_t, q, wstu, a1, wisk,
           a3, wski, a4, wisu, a2, ams, amk, bu):
    U, I = indicator.shape
    D = user_t.shape[1]
    S = skill_t.shape[0]
    nq = 5
    hb = bu // nq
    grid = (U // bu,)
    cst = lambda u: (0, 0)
    return pl.pallas_call(
        _dense_body,
        grid=grid,
        in_specs=[
            pl.BlockSpec((hb, I),
                         (lambda jj: (lambda u: (nq * u + jj, 0)))(j))
            for j in range(nq)
        ] + [
            pl.BlockSpec((bu, D), lambda u: (u, 0)),
            pl.BlockSpec((I, D), cst),
            pl.BlockSpec((S, D), cst),
            pl.BlockSpec((I, D), cst),
        ] + [pl.BlockSpec(w.shape, cst) for w in
             (wstu, a1, wisk, a3, wski, a4, wisu, a2, ams, amk)],
        out_specs=[
            pl.BlockSpec((bu, D), lambda u: (u, 0)),
            pl.BlockSpec((S, D), cst),
            pl.BlockSpec((I, 2 * D), cst),
        ],
        out_shape=[
            jax.ShapeDtypeStruct((U, D), jnp.float32),
            jax.ShapeDtypeStruct((S, D), jnp.float32),
            jax.ShapeDtypeStruct((I, 2 * D), jnp.float32),
        ],
        scratch_shapes=[
            pltpu.VMEM((I, 2 * D), jnp.bfloat16),
            pltpu.VMEM((I, D), jnp.float32),
            pltpu.VMEM((I, 2 * D), jnp.float32),
        ],
    )(indicator, indicator, indicator, indicator, indicator,
      user_t, item_t, skill_t, q, wstu, a1,
      wisk, a3, wski, a4, wisu, a2, ams, amk)


def _sc_gather(fu, packed_item, uidx, iidx):
    B = uidx.shape[0]
    D = fu.shape[1]
    D2 = packed_item.shape[1]
    info = plsc.get_sparse_core_info()
    nw = info.num_cores * info.num_subcores
    bpw = B // nw
    mesh = plsc.VectorSubcoreMesh(core_axis_name="c", subcore_axis_name="s")

    @functools.partial(
        pl.kernel,
        mesh=mesh,
        out_type=[
            jax.ShapeDtypeStruct((B, D), fu.dtype),
            jax.ShapeDtypeStruct((B, D2), packed_item.dtype),
        ],
        scratch_types=[
            pltpu.VMEM((bpw,), jnp.int32),
            pltpu.VMEM((bpw, D), fu.dtype),
            pltpu.VMEM((bpw,), jnp.int32),
            pltpu.VMEM((bpw, D2), packed_item.dtype),
            pltpu.SemaphoreType.DMA,
            pltpu.SemaphoreType.DMA,
        ],
    )
    def k(fu_hbm, pit_hbm, uidx_hbm, iidx_hbm, ue_hbm, ie_hbm,
          uix_v, urows_v, iix_v, irows_v, sem_u, sem_i):
        wid = lax.axis_index("s") * info.num_cores + lax.axis_index("c")
        base = wid * bpw
        pltpu.sync_copy(uidx_hbm.at[pl.ds(base, bpw)], uix_v)
        pltpu.sync_copy(iidx_hbm.at[pl.ds(base, bpw)], iix_v)
        cp_u = pltpu.async_copy(fu_hbm.at[uix_v], urows_v, sem_u)
        cp_i = pltpu.async_copy(pit_hbm.at[iix_v], irows_v, sem_i)
        cp_u.wait()
        cp_i.wait()
        pltpu.sync_copy(urows_v, ue_hbm.at[pl.ds(base, bpw)])
        pltpu.sync_copy(irows_v, ie_hbm.at[pl.ds(base, bpw)])

    return k(fu, packed_item, uidx, iidx)


def _pred_body(ue_ref, iep_ref, fsk_ref, wfs_ref, bfs, wfi_ref,
               bfi, wpred, bpred, out_ref):
    ue = ue_ref[...]
    iep = iep_ref[...]
    ie = iep[:, :128]
    qb = iep[:, 128:]
    wfs = wfs_ref[...]
    wfi = wfi_ref[...]
    dnT = (((1,), (1,)), ((), ()))
    se_num = jnp.dot(qb, fsk_ref[...], preferred_element_type=jnp.float32)
    se = se_num / (jnp.sum(qb, axis=1, keepdims=True) + EPS)
    hs = jax.nn.sigmoid(
        lax.dot_general(ue, wfs[:, :128], dnT,
                        preferred_element_type=jnp.float32)
        + lax.dot_general(se, wfs[:, 128:], dnT,
                          preferred_element_type=jnp.float32)
        + bfs[...])
    hi = jax.nn.sigmoid(
        lax.dot_general(ie, wfi[:, :128], dnT,
                        preferred_element_type=jnp.float32)
        + lax.dot_general(se, wfi[:, 128:], dnT,
                          preferred_element_type=jnp.float32)
        + bfi[...])
    z = jnp.sum((hs - hi) * wpred[...], axis=1, keepdims=True) + bpred[...]
    out_ref[...] = jax.nn.sigmoid(z)


def _predict(ue, iep, fsk, wfs, bfs, wfi, bfi, wpred, bpred):
    B = ue.shape[0]
    return pl.pallas_call(
        _pred_body,
        out_shape=jax.ShapeDtypeStruct((B, 1), jnp.float32),
    )(ue, iep, fsk, wfs, bfs, wfi, bfi, wpred, bpred)


def kernel(user, item, q, indicator, user_table, item_table, skill_table,
           W_stu, a_stu, W_item_stu, W_item_skill, a_item_stu, a_item_skill,
           a_map_stu, a_map_skill, W_skill_item, a_skill_item, W_fuse_stu,
           b_fuse_stu, W_fuse_item, b_fuse_item, W_pred, b_pred):
    d = user_table.shape[1]
    r2 = lambda v: v.reshape(1, 2 * d)

    fu, fsk, packed_item = _dense(
        indicator, user_table, item_table, skill_table, q,
        W_stu, r2(a_stu), W_item_skill, r2(a_item_skill),
        W_skill_item, r2(a_skill_item), W_item_stu, r2(a_item_stu),
        r2(a_map_stu), r2(a_map_skill), bu=2000)

    ue, iep = _sc_gather(fu, packed_item, user.astype(jnp.int32),
                         item.astype(jnp.int32))

    pred = _predict(ue, iep, fsk,
                    W_fuse_stu, b_fuse_stu.reshape(1, d),
                    W_fuse_item, b_fuse_item.reshape(1, d),
                    W_pred, b_pred.reshape(1, 1))
    return pred.reshape(-1)
